# Initial kernel scaffold; baseline (speedup 1.0000x reference)
import jax
import jax.numpy as jnp
from jax.experimental import pallas as pl


def kernel(x, n_id, src_n_id, dst_n_id, edge_index, edge_attr, t, his_edge_index, enc_t_table, z, Wq, bq, Wk, bk, Wv, bv, We, Ws, bs):
    raise NotImplementedError("write your pallas kernel here")


# speedup vs baseline: 1.1395x; 1.1395x over previous
import functools

import jax
import jax.numpy as jnp
from jax.experimental import pallas as pl
from jax.experimental.pallas import tpu as pltpu

N = 100000
NODE_DIM = 10
EDGE_DIM = 16
EMB = 20
TIME = 10
E = 1600000

_NODE_BLK = 2000
_EDGE_BLK = 8000


def _node_prep_body(xe_ref, z_ref, Csrc_ref, Cdst_ref, WqT_ref, WkT_ref,
                    WvT_ref, WsT_ref, b_ref, out_ref):
    xe = xe_ref[...]
    z = z_ref[...]
    As = jnp.dot(xe, Csrc_ref[...], preferred_element_type=jnp.float32)
    Ad = jnp.dot(xe, Cdst_ref[...], preferred_element_type=jnp.float32)
    q = jnp.dot(z, WqT_ref[...], preferred_element_type=jnp.float32) + b_ref[0:1, :]
    k = jnp.dot(z, WkT_ref[...], preferred_element_type=jnp.float32) + b_ref[1:2, :]
    v = jnp.dot(z, WvT_ref[...], preferred_element_type=jnp.float32) + b_ref[2:3, :]
    skip = jnp.dot(z, WsT_ref[...], preferred_element_type=jnp.float32) + b_ref[3:4, :]
    out_ref[...] = jnp.concatenate([As, Ad, q, k, v, skip], axis=-1)


def _node_prep(xe, z, Csrc, Cdst, WqT, WkT, WvT, WsT, b4):
    grid = (N // _NODE_BLK,)
    out = pl.pallas_call(
        _node_prep_body,
        grid=grid,
        in_specs=[
            pl.BlockSpec((_NODE_BLK, EMB), lambda i: (i, 0)),
            pl.BlockSpec((_NODE_BLK, EMB), lambda i: (i, 0)),
            pl.BlockSpec((EMB, EMB), lambda i: (0, 0)),
            pl.BlockSpec((EMB, EMB), lambda i: (0, 0)),
            pl.BlockSpec((EMB, EMB), lambda i: (0, 0)),
            pl.BlockSpec((EMB, EMB), lambda i: (0, 0)),
            pl.BlockSpec((EMB, EMB), lambda i: (0, 0)),
            pl.BlockSpec((EMB, EMB), lambda i: (0, 0)),
            pl.BlockSpec((4, EMB), lambda i: (0, 0)),
        ],
        out_specs=pl.BlockSpec((_NODE_BLK, 6 * EMB), lambda i: (i, 0)),
        out_shape=jax.ShapeDtypeStruct((N, 6 * EMB), jnp.float32),
    )(xe, z, Csrc, Cdst, WqT, WkT, WvT, WsT, b4)
    return out


def _edge_prep_body(ea_ref, t_ref, W0T_ref, s12_ref, out_ref):
    ea = ea_ref[...]
    t = t_ref[...]
    out_ref[...] = (
        jnp.dot(ea, W0T_ref[...], preferred_element_type=jnp.float32)
        + t * s12_ref[...]
    )


def _edge_prep(edge_attr, t, W0T, s12):
    grid = (E // _EDGE_BLK,)
    out = pl.pallas_call(
        _edge_prep_body,
        grid=grid,
        in_specs=[
            pl.BlockSpec((_EDGE_BLK, EDGE_DIM), lambda i: (i, 0)),
            pl.BlockSpec((_EDGE_BLK, 1), lambda i: (i, 0)),
            pl.BlockSpec((EDGE_DIM, EMB), lambda i: (0, 0)),
            pl.BlockSpec((1, EMB), lambda i: (0, 0)),
        ],
        out_specs=pl.BlockSpec((_EDGE_BLK, EMB), lambda i: (i, 0)),
        out_shape=jax.ShapeDtypeStruct((E, EMB), jnp.float32),
    )(edge_attr, t, W0T, s12)
    return out


def kernel(x, n_id, src_n_id, dst_n_id, edge_index, edge_attr, t,
           his_edge_index, enc_t_table, z,
           Wq, bq, Wk, bk, Wv, bv, We, Ws, bs):
    m_ei = jnp.concatenate([edge_index, his_edge_index], axis=1)
    m_src, m_dst = m_ei[0], m_ei[1]
    x3 = x
    for _ in range(3):
        x3 = jax.ops.segment_sum(x3[m_src], m_dst, num_segments=N)

    W0 = We[:, :EDGE_DIM]
    W1 = We[:, EDGE_DIM:EDGE_DIM + TIME]
    W2 = We[:, EDGE_DIM + TIME:EDGE_DIM + 2 * TIME]
    W3 = We[:, EDGE_DIM + 2 * TIME:EDGE_DIM + 2 * TIME + NODE_DIM]
    W4 = We[:, EDGE_DIM + 2 * TIME + NODE_DIM:]
    s12 = jnp.sum(W1 + W2, axis=1)[None, :]
    Csrc = jnp.concatenate([W3.T, -W1.T], axis=0)
    Cdst = jnp.concatenate([W4.T, -W2.T], axis=0)
    xe = jnp.concatenate([x3, enc_t_table], axis=1)
    b4 = jnp.stack([bq, bk, bv, bs], axis=0)

    nt = _node_prep(xe, z, Csrc, Cdst, Wq.T, Wk.T, Wv.T, Ws.T, b4)
    As, Ad, q, k, v, skip = (nt[:, i * EMB:(i + 1) * EMB] for i in range(6))
    EA20 = _edge_prep(edge_attr, t[:, None], W0.T, s12)

    src, dst = edge_index[0], edge_index[1]
    e = EA20 + As[src_n_id] + Ad[dst_n_id]
    k_j = k[src] + e
    alpha = jnp.sum(q[dst] * k_j, axis=-1) / jnp.sqrt(float(EMB))
    amax = jax.ops.segment_max(alpha, dst, num_segments=N)
    amax = jnp.where(jnp.isfinite(amax), amax, 0.0)
    al = jnp.exp(alpha - amax[dst])
    den = jax.ops.segment_sum(al, dst, num_segments=N)
    msg = (v[src] + e) * al[:, None]
    out_pre = jax.ops.segment_sum(msg, dst, num_segments=N)
    out = jnp.where(den[:, None] != 0.0, out_pre / den[:, None], 0.0) + skip
    return out
